# Initial kernel scaffold; baseline (speedup 1.0000x reference)
import jax
import jax.numpy as jnp
from jax.experimental import pallas as pl


def kernel(x, edge_index, batch, W1_rel, W1_root, b1, W2_rel, W2_root, b2, W3_rel, W3_root, b3, Wl, bl):
    raise NotImplementedError("write your pallas kernel here")


# speedup vs baseline: 5.7606x; 5.7606x over previous
import functools

import jax
import jax.numpy as jnp
from jax import lax
from jax.experimental import pallas as pl
from jax.experimental.pallas import tpu as pltpu
from jax.experimental.pallas import tpu_sc as plsc

N = 10000
E = 320000
H = 128
G = 64
C = 10

NC = 2
NS = 16
HH = H // NC
EPT = E // NS
K = 125
CHT = EPT // K
RB = 10
BN = N // RB
NP = 10240
RPT = NP // NS
ZR = 128


def _scatter_body(m_hbm, src_hbm, dst_hbm, zeros_hbm, out_hbm,
                  src_v, dst_v, rows_v, zbuf, agg, sem):
    c = lax.axis_index("c")
    s = lax.axis_index("s")

    pltpu.sync_copy(zeros_hbm, zbuf)
    for j in range(RPT // ZR):
        row0 = pl.multiple_of(s * RPT + j * ZR, ZR)
        pltpu.sync_copy(zbuf, agg.at[pl.ds(row0, ZR)])
    plsc.subcore_barrier()

    pltpu.sync_copy(src_hbm.at[s], src_v)
    pltpu.sync_copy(dst_hbm.at[s], dst_v)

    def body(i, carry):
        pltpu.async_copy(m_hbm.at[c].at[src_v.at[i]], rows_v, sem).wait()
        pltpu.sync_copy(rows_v, agg.at[dst_v.at[i]], add=True)
        return carry

    lax.fori_loop(0, CHT, body, 0)
    plsc.subcore_barrier()

    for j in range(RPT // ZR):
        row0 = pl.multiple_of(s * RPT + j * ZR, ZR)
        pltpu.sync_copy(agg.at[pl.ds(row0, ZR)], zbuf)
        pltpu.sync_copy(zbuf, out_hbm.at[c, pl.ds(row0, ZR)])


@functools.lru_cache(maxsize=None)
def _make_sc_scatter():
    return pl.kernel(
        _scatter_body,
        out_type=jax.ShapeDtypeStruct((NC, NP, HH), jnp.float32),
        mesh=plsc.VectorSubcoreMesh(core_axis_name="c", subcore_axis_name="s",
                                    num_cores=NC, num_subcores=NS),
        scratch_types=[
            pltpu.VMEM((CHT, K), jnp.int32),
            pltpu.VMEM((CHT, K), jnp.int32),
            pltpu.VMEM((K, HH), jnp.float32),
            pltpu.VMEM((ZR, HH), jnp.float32),
            pltpu.VMEM_SHARED((NP, HH), jnp.float32),
            pltpu.SemaphoreType.DMA,
        ],
        compiler_params=pltpu.CompilerParams(use_tc_tiling_on_sc=False),
    )


def _lin2_body(h_ref, wr_ref, wo_ref, b_ref, m_ref, r_ref):
    h = h_ref[...]
    m = jnp.dot(h, wr_ref[...], preferred_element_type=jnp.float32)
    m_ref[0] = m[:, :HH]
    m_ref[1] = m[:, HH:]
    r_ref[...] = (jnp.dot(h, wo_ref[...], preferred_element_type=jnp.float32)
                  + b_ref[...])


_lin2 = pl.pallas_call(
    _lin2_body,
    grid=(RB,),
    in_specs=[
        pl.BlockSpec((BN, H), lambda i: (i, 0)),
        pl.BlockSpec((H, H), lambda i: (0, 0)),
        pl.BlockSpec((H, H), lambda i: (0, 0)),
        pl.BlockSpec((1, H), lambda i: (0, 0)),
    ],
    out_specs=[pl.BlockSpec((NC, BN, HH), lambda i: (0, i, 0)),
               pl.BlockSpec((BN, H), lambda i: (i, 0))],
    out_shape=[jax.ShapeDtypeStruct((NC, N, HH), jnp.float32),
               jax.ShapeDtypeStruct((N, H), jnp.float32)],
)


def _comb_lin2_body(p_ref, rp_ref, wr_ref, wo_ref, b_ref, m_ref, r_ref):
    agg = jnp.concatenate([p_ref[0], p_ref[1]], axis=1)
    h = jnp.maximum(agg + rp_ref[...], 0.0)
    m = jnp.dot(h, wr_ref[...], preferred_element_type=jnp.float32)
    m_ref[0] = m[:, :HH]
    m_ref[1] = m[:, HH:]
    r_ref[...] = (jnp.dot(h, wo_ref[...], preferred_element_type=jnp.float32)
                  + b_ref[...])


_comb_lin2 = pl.pallas_call(
    _comb_lin2_body,
    grid=(RB,),
    in_specs=[
        pl.BlockSpec((NC, BN, HH), lambda i: (0, i, 0)),
        pl.BlockSpec((BN, H), lambda i: (i, 0)),
        pl.BlockSpec((H, H), lambda i: (0, 0)),
        pl.BlockSpec((H, H), lambda i: (0, 0)),
        pl.BlockSpec((1, H), lambda i: (0, 0)),
    ],
    out_specs=[pl.BlockSpec((NC, BN, HH), lambda i: (0, i, 0)),
               pl.BlockSpec((BN, H), lambda i: (i, 0))],
    out_shape=[jax.ShapeDtypeStruct((NC, N, HH), jnp.float32),
               jax.ShapeDtypeStruct((N, H), jnp.float32)],
)


def _pool_body(p_ref, rp_ref, batch_ref, wl_ref, bl_ref,
               pooled_ref, out_ref, sum_acc, cnt_acc):
    i = pl.program_id(0)
    agg = jnp.concatenate([p_ref[0], p_ref[1]], axis=1)
    h = agg + rp_ref[...]
    b_row = batch_ref[0]
    oh_t = (lax.broadcasted_iota(jnp.int32, (G, BN), 0) == b_row
            ).astype(jnp.float32)

    @pl.when(i == 0)
    def _():
        sum_acc[...] = jnp.zeros_like(sum_acc)
        cnt_acc[...] = jnp.zeros_like(cnt_acc)

    sum_acc[...] += lax.dot_general(oh_t, h, (((1,), (0,)), ((), ())),
                                    preferred_element_type=jnp.float32)
    cnt_acc[...] += jnp.sum(oh_t, axis=1)[:, None]

    @pl.when(i == RB - 1)
    def _():
        pooled = sum_acc[...] / jnp.maximum(cnt_acc[...], 1.0)
        pooled_ref[...] = pooled
        out_ref[...] = (jnp.dot(pooled, wl_ref[...],
                                preferred_element_type=jnp.float32)
                        + bl_ref[...])


_pool = pl.pallas_call(
    _pool_body,
    grid=(RB,),
    in_specs=[
        pl.BlockSpec((NC, BN, HH), lambda i: (0, i, 0)),
        pl.BlockSpec((BN, H), lambda i: (i, 0)),
        pl.BlockSpec((1, 1, BN), lambda i: (i, 0, 0)),
        pl.BlockSpec((H, C), lambda i: (0, 0)),
        pl.BlockSpec((1, C), lambda i: (0, 0)),
    ],
    out_specs=[pl.BlockSpec((G, H), lambda i: (0, 0)),
               pl.BlockSpec((G, C), lambda i: (0, 0))],
    out_shape=[jax.ShapeDtypeStruct((G, H), jnp.float32),
               jax.ShapeDtypeStruct((G, C), jnp.float32)],
    scratch_shapes=[pltpu.VMEM((G, H), jnp.float32),
                    pltpu.VMEM((G, 1), jnp.float32)],
)


def kernel(x, edge_index, batch, W1_rel, W1_root, b1,
           W2_rel, W2_root, b2, W3_rel, W3_root, b3, Wl, bl):
    src = edge_index[0].reshape(NS, CHT, K)
    dst = edge_index[1].reshape(NS, CHT, K)
    zeros = jnp.zeros((ZR, HH), jnp.float32)
    batch3 = batch.reshape(RB, 1, BN)

    sc_scatter = _make_sc_scatter()
    m1, r1 = _lin2(x, W1_rel, W1_root, b1.reshape(1, H))
    p = sc_scatter(m1, src, dst, zeros)
    m2, r2 = _comb_lin2(p, r1, W2_rel, W2_root, b2.reshape(1, H))
    p = sc_scatter(m2, src, dst, zeros)
    m3, r3 = _comb_lin2(p, r2, W3_rel, W3_root, b3.reshape(1, H))
    p = sc_scatter(m3, src, dst, zeros)
    pooled, out = _pool(p, r3, batch3, Wl, bl.reshape(1, C))
    return (pooled, out)


# speedup vs baseline: 7.1847x; 1.2472x over previous
import functools

import jax
import jax.numpy as jnp
from jax import lax
from jax.experimental import pallas as pl
from jax.experimental.pallas import tpu as pltpu
from jax.experimental.pallas import tpu_sc as plsc

N = 10000
E = 320000
H = 128
G = 64
C = 10

NC = 2
NS = 16
HH = H // NC
EPT = E // NS
K = 125
CHT = EPT // K
RB = 10
BN = N // RB
NP = 10240
RPT = NP // NS
ZR = 128


def _scatter_body(m_hbm, src_hbm, dst_hbm, zeros_hbm, out_hbm,
                  src_v, dst_v, rows_v0, rows_v1, zbuf, agg,
                  gsem0, gsem1, ssem0, ssem1):
    c = lax.axis_index("c")
    s = lax.axis_index("s")

    pltpu.sync_copy(zeros_hbm, zbuf)
    for j in range(RPT // ZR):
        row0 = pl.multiple_of(s * RPT + j * ZR, ZR)
        pltpu.sync_copy(zbuf, agg.at[pl.ds(row0, ZR)])
    plsc.subcore_barrier()

    pltpu.sync_copy(src_hbm.at[s], src_v)
    pltpu.sync_copy(dst_hbm.at[s], dst_v)

    plane = m_hbm.at[c]
    rows = (rows_v0, rows_v1)
    gsem = (gsem0, gsem1)
    ssem = (ssem0, ssem1)

    def gather(i, b):
        return pltpu.make_async_copy(plane.at[src_v.at[i]], rows[b], gsem[b])

    def scatter(i, b):
        return pltpu.make_async_copy(rows[b], agg.at[dst_v.at[i]], ssem[b])

    gather(0, 0).start()
    gather(0, 0).wait()
    scatter(0, 0).start(add=True)
    gather(1, 1).start()

    def body(it, carry):
        i = 2 * it
        gather(i + 1, 1).wait()
        scatter(i + 1, 1).start(add=True)
        scatter(i, 0).wait()
        gather(i + 2, 0).start()
        gather(i + 2, 0).wait()
        scatter(i + 2, 0).start(add=True)
        scatter(i + 1, 1).wait()
        gather(i + 3, 1).start()
        return carry

    lax.fori_loop(0, CHT // 2 - 1, body, 0)
    gather(CHT - 1, 1).wait()
    scatter(CHT - 1, 1).start(add=True)
    scatter(CHT - 2, 0).wait()
    scatter(CHT - 1, 1).wait()
    plsc.subcore_barrier()

    for j in range(RPT // ZR):
        row0 = pl.multiple_of(s * RPT + j * ZR, ZR)
        pltpu.sync_copy(agg.at[pl.ds(row0, ZR)], zbuf)
        pltpu.sync_copy(zbuf, out_hbm.at[c, pl.ds(row0, ZR)])


@functools.lru_cache(maxsize=None)
def _make_sc_scatter():
    return pl.kernel(
        _scatter_body,
        out_type=jax.ShapeDtypeStruct((NC, NP, HH), jnp.float32),
        mesh=plsc.VectorSubcoreMesh(core_axis_name="c", subcore_axis_name="s",
                                    num_cores=NC, num_subcores=NS),
        scratch_types=[
            pltpu.VMEM((CHT, K), jnp.int32),
            pltpu.VMEM((CHT, K), jnp.int32),
            pltpu.VMEM((K, HH), jnp.float32),
            pltpu.VMEM((K, HH), jnp.float32),
            pltpu.VMEM((ZR, HH), jnp.float32),
            pltpu.VMEM_SHARED((NP, HH), jnp.float32),
            pltpu.SemaphoreType.DMA,
            pltpu.SemaphoreType.DMA,
            pltpu.SemaphoreType.DMA,
            pltpu.SemaphoreType.DMA,
        ],
        compiler_params=pltpu.CompilerParams(use_tc_tiling_on_sc=False),
    )


def _lin2_body(h_ref, wr_ref, wo_ref, b_ref, m_ref, r_ref):
    h = h_ref[...]
    m = jnp.dot(h, wr_ref[...], preferred_element_type=jnp.float32)
    m_ref[0] = m[:, :HH]
    m_ref[1] = m[:, HH:]
    r_ref[...] = (jnp.dot(h, wo_ref[...], preferred_element_type=jnp.float32)
                  + b_ref[...])


_lin2 = pl.pallas_call(
    _lin2_body,
    grid=(RB,),
    in_specs=[
        pl.BlockSpec((BN, H), lambda i: (i, 0)),
        pl.BlockSpec((H, H), lambda i: (0, 0)),
        pl.BlockSpec((H, H), lambda i: (0, 0)),
        pl.BlockSpec((1, H), lambda i: (0, 0)),
    ],
    out_specs=[pl.BlockSpec((NC, BN, HH), lambda i: (0, i, 0)),
               pl.BlockSpec((BN, H), lambda i: (i, 0))],
    out_shape=[jax.ShapeDtypeStruct((NC, N, HH), jnp.float32),
               jax.ShapeDtypeStruct((N, H), jnp.float32)],
)


def _comb_lin2_body(p_ref, rp_ref, wr_ref, wo_ref, b_ref, m_ref, r_ref):
    agg = jnp.concatenate([p_ref[0], p_ref[1]], axis=1)
    h = jnp.maximum(agg + rp_ref[...], 0.0)
    m = jnp.dot(h, wr_ref[...], preferred_element_type=jnp.float32)
    m_ref[0] = m[:, :HH]
    m_ref[1] = m[:, HH:]
    r_ref[...] = (jnp.dot(h, wo_ref[...], preferred_element_type=jnp.float32)
                  + b_ref[...])


_comb_lin2 = pl.pallas_call(
    _comb_lin2_body,
    grid=(RB,),
    in_specs=[
        pl.BlockSpec((NC, BN, HH), lambda i: (0, i, 0)),
        pl.BlockSpec((BN, H), lambda i: (i, 0)),
        pl.BlockSpec((H, H), lambda i: (0, 0)),
        pl.BlockSpec((H, H), lambda i: (0, 0)),
        pl.BlockSpec((1, H), lambda i: (0, 0)),
    ],
    out_specs=[pl.BlockSpec((NC, BN, HH), lambda i: (0, i, 0)),
               pl.BlockSpec((BN, H), lambda i: (i, 0))],
    out_shape=[jax.ShapeDtypeStruct((NC, N, HH), jnp.float32),
               jax.ShapeDtypeStruct((N, H), jnp.float32)],
)


def _pool_body(p_ref, rp_ref, batch_ref, wl_ref, bl_ref,
               pooled_ref, out_ref, sum_acc, cnt_acc):
    i = pl.program_id(0)
    agg = jnp.concatenate([p_ref[0], p_ref[1]], axis=1)
    h = agg + rp_ref[...]
    b_row = batch_ref[0]
    oh_t = (lax.broadcasted_iota(jnp.int32, (G, BN), 0) == b_row
            ).astype(jnp.float32)

    @pl.when(i == 0)
    def _():
        sum_acc[...] = jnp.zeros_like(sum_acc)
        cnt_acc[...] = jnp.zeros_like(cnt_acc)

    sum_acc[...] += lax.dot_general(oh_t, h, (((1,), (0,)), ((), ())),
                                    preferred_element_type=jnp.float32)
    cnt_acc[...] += jnp.sum(oh_t, axis=1)[:, None]

    @pl.when(i == RB - 1)
    def _():
        pooled = sum_acc[...] / jnp.maximum(cnt_acc[...], 1.0)
        pooled_ref[...] = pooled
        out_ref[...] = (jnp.dot(pooled, wl_ref[...],
                                preferred_element_type=jnp.float32)
                        + bl_ref[...])


_pool = pl.pallas_call(
    _pool_body,
    grid=(RB,),
    in_specs=[
        pl.BlockSpec((NC, BN, HH), lambda i: (0, i, 0)),
        pl.BlockSpec((BN, H), lambda i: (i, 0)),
        pl.BlockSpec((1, 1, BN), lambda i: (i, 0, 0)),
        pl.BlockSpec((H, C), lambda i: (0, 0)),
        pl.BlockSpec((1, C), lambda i: (0, 0)),
    ],
    out_specs=[pl.BlockSpec((G, H), lambda i: (0, 0)),
               pl.BlockSpec((G, C), lambda i: (0, 0))],
    out_shape=[jax.ShapeDtypeStruct((G, H), jnp.float32),
               jax.ShapeDtypeStruct((G, C), jnp.float32)],
    scratch_shapes=[pltpu.VMEM((G, H), jnp.float32),
                    pltpu.VMEM((G, 1), jnp.float32)],
)


def kernel(x, edge_index, batch, W1_rel, W1_root, b1,
           W2_rel, W2_root, b2, W3_rel, W3_root, b3, Wl, bl):
    src = edge_index[0].reshape(NS, CHT, K)
    dst = edge_index[1].reshape(NS, CHT, K)
    zeros = jnp.zeros((ZR, HH), jnp.float32)
    batch3 = batch.reshape(RB, 1, BN)

    sc_scatter = _make_sc_scatter()
    m1, r1 = _lin2(x, W1_rel, W1_root, b1.reshape(1, H))
    p = sc_scatter(m1, src, dst, zeros)
    m2, r2 = _comb_lin2(p, r1, W2_rel, W2_root, b2.reshape(1, H))
    p = sc_scatter(m2, src, dst, zeros)
    m3, r3 = _comb_lin2(p, r2, W3_rel, W3_root, b3.reshape(1, H))
    p = sc_scatter(m3, src, dst, zeros)
    pooled, out = _pool(p, r3, batch3, Wl, bl.reshape(1, C))
    return (pooled, out)


# speedup vs baseline: 9.4691x; 1.3179x over previous
import functools

import jax
import jax.numpy as jnp
from jax import lax
from jax.experimental import pallas as pl
from jax.experimental.pallas import tpu as pltpu
from jax.experimental.pallas import tpu_sc as plsc

N = 10000
E = 320000
H = 128
G = 64
C = 10

NC = 2
NS = 16
HH = H // NC
EPT = E // NS
K = 125
CHT = EPT // K
RB = 10
BN = N // RB
NP = 10240
RPT = NP // NS
ZR = 128


NB = 4
LA = NB // 2


def _scatter_body(m_hbm, src_hbm, dst_hbm, zeros_hbm, out_hbm, *scr):
    src_v, dst_v = scr[0], scr[1]
    rows = scr[2:2 + NB]
    zbuf, agg = scr[2 + NB], scr[3 + NB]
    gsem = scr[4 + NB:4 + 2 * NB]
    ssem = scr[4 + 2 * NB:4 + 3 * NB]
    c = lax.axis_index("c")
    s = lax.axis_index("s")

    pltpu.sync_copy(zeros_hbm, zbuf)
    for j in range(RPT // ZR):
        row0 = pl.multiple_of(s * RPT + j * ZR, ZR)
        pltpu.sync_copy(zbuf, agg.at[pl.ds(row0, ZR)])
    plsc.subcore_barrier()

    pltpu.sync_copy(src_hbm.at[s], src_v)
    pltpu.sync_copy(dst_hbm.at[s], dst_v)

    plane = m_hbm.at[c]

    def gather(i, b):
        return pltpu.make_async_copy(plane.at[src_v.at[i]], rows[b], gsem[b])

    def scatter(i, b):
        return pltpu.make_async_copy(rows[b], agg.at[dst_v.at[i]], ssem[b])

    for i in range(LA):
        gather(i, i % NB).start()
    for i in range(LA):
        gather(i, i % NB).wait()
        scatter(i, i % NB).start(add=True)
        gather(i + LA, (i + LA) % NB).start()

    def body(it, _):
        for b in range(NB):
            i = LA + NB * it + b
            bb = (LA + b) % NB
            gather(i, bb).wait()
            scatter(i, bb).start(add=True)
            scatter(i - LA, (bb + LA) % NB).wait()
            gather(i + LA, (bb + LA) % NB).start()
        return _

    lax.fori_loop(0, (CHT - 2 * LA) // NB, body, 0)
    for i in range(CHT - LA, CHT):
        gather(i, i % NB).wait()
        scatter(i, i % NB).start(add=True)
        scatter(i - LA, (i - LA) % NB).wait()
    for i in range(CHT - LA, CHT):
        scatter(i, i % NB).wait()
    plsc.subcore_barrier()

    for j in range(RPT // ZR):
        row0 = pl.multiple_of(s * RPT + j * ZR, ZR)
        pltpu.sync_copy(agg.at[pl.ds(row0, ZR)], zbuf)
        pltpu.sync_copy(zbuf, out_hbm.at[c, pl.ds(row0, ZR)])


@functools.lru_cache(maxsize=None)
def _make_sc_scatter():
    return pl.kernel(
        _scatter_body,
        out_type=jax.ShapeDtypeStruct((NC, NP, HH), jnp.float32),
        mesh=plsc.VectorSubcoreMesh(core_axis_name="c", subcore_axis_name="s",
                                    num_cores=NC, num_subcores=NS),
        scratch_types=(
            [pltpu.VMEM((CHT, K), jnp.int32)] * 2
            + [pltpu.VMEM((K, HH), jnp.float32)] * NB
            + [pltpu.VMEM((ZR, HH), jnp.float32),
               pltpu.VMEM_SHARED((NP, HH), jnp.float32)]
            + [pltpu.SemaphoreType.DMA] * (2 * NB)
        ),
        compiler_params=pltpu.CompilerParams(use_tc_tiling_on_sc=False),
    )


def _lin2_body(h_ref, wr_ref, wo_ref, b_ref, m_ref, r_ref):
    h = h_ref[...]
    m = jnp.dot(h, wr_ref[...], preferred_element_type=jnp.float32)
    m_ref[0] = m[:, :HH]
    m_ref[1] = m[:, HH:]
    r_ref[...] = (jnp.dot(h, wo_ref[...], preferred_element_type=jnp.float32)
                  + b_ref[...])


_lin2 = pl.pallas_call(
    _lin2_body,
    grid=(RB,),
    in_specs=[
        pl.BlockSpec((BN, H), lambda i: (i, 0)),
        pl.BlockSpec((H, H), lambda i: (0, 0)),
        pl.BlockSpec((H, H), lambda i: (0, 0)),
        pl.BlockSpec((1, H), lambda i: (0, 0)),
    ],
    out_specs=[pl.BlockSpec((NC, BN, HH), lambda i: (0, i, 0)),
               pl.BlockSpec((BN, H), lambda i: (i, 0))],
    out_shape=[jax.ShapeDtypeStruct((NC, N, HH), jnp.float32),
               jax.ShapeDtypeStruct((N, H), jnp.float32)],
)


def _comb_lin2_body(p_ref, rp_ref, wr_ref, wo_ref, b_ref, m_ref, r_ref):
    agg = jnp.concatenate([p_ref[0], p_ref[1]], axis=1)
    h = jnp.maximum(agg + rp_ref[...], 0.0)
    m = jnp.dot(h, wr_ref[...], preferred_element_type=jnp.float32)
    m_ref[0] = m[:, :HH]
    m_ref[1] = m[:, HH:]
    r_ref[...] = (jnp.dot(h, wo_ref[...], preferred_element_type=jnp.float32)
                  + b_ref[...])


_comb_lin2 = pl.pallas_call(
    _comb_lin2_body,
    grid=(RB,),
    in_specs=[
        pl.BlockSpec((NC, BN, HH), lambda i: (0, i, 0)),
        pl.BlockSpec((BN, H), lambda i: (i, 0)),
        pl.BlockSpec((H, H), lambda i: (0, 0)),
        pl.BlockSpec((H, H), lambda i: (0, 0)),
        pl.BlockSpec((1, H), lambda i: (0, 0)),
    ],
    out_specs=[pl.BlockSpec((NC, BN, HH), lambda i: (0, i, 0)),
               pl.BlockSpec((BN, H), lambda i: (i, 0))],
    out_shape=[jax.ShapeDtypeStruct((NC, N, HH), jnp.float32),
               jax.ShapeDtypeStruct((N, H), jnp.float32)],
)


def _pool_body(p_ref, rp_ref, batch_ref, wl_ref, bl_ref,
               pooled_ref, out_ref, sum_acc, cnt_acc):
    i = pl.program_id(0)
    agg = jnp.concatenate([p_ref[0], p_ref[1]], axis=1)
    h = agg + rp_ref[...]
    b_row = batch_ref[0]
    oh_t = (lax.broadcasted_iota(jnp.int32, (G, BN), 0) == b_row
            ).astype(jnp.float32)

    @pl.when(i == 0)
    def _():
        sum_acc[...] = jnp.zeros_like(sum_acc)
        cnt_acc[...] = jnp.zeros_like(cnt_acc)

    sum_acc[...] += lax.dot_general(oh_t, h, (((1,), (0,)), ((), ())),
                                    preferred_element_type=jnp.float32)
    cnt_acc[...] += jnp.sum(oh_t, axis=1)[:, None]

    @pl.when(i == RB - 1)
    def _():
        pooled = sum_acc[...] / jnp.maximum(cnt_acc[...], 1.0)
        pooled_ref[...] = pooled
        out_ref[...] = (jnp.dot(pooled, wl_ref[...],
                                preferred_element_type=jnp.float32)
                        + bl_ref[...])


_pool = pl.pallas_call(
    _pool_body,
    grid=(RB,),
    in_specs=[
        pl.BlockSpec((NC, BN, HH), lambda i: (0, i, 0)),
        pl.BlockSpec((BN, H), lambda i: (i, 0)),
        pl.BlockSpec((1, 1, BN), lambda i: (i, 0, 0)),
        pl.BlockSpec((H, C), lambda i: (0, 0)),
        pl.BlockSpec((1, C), lambda i: (0, 0)),
    ],
    out_specs=[pl.BlockSpec((G, H), lambda i: (0, 0)),
               pl.BlockSpec((G, C), lambda i: (0, 0))],
    out_shape=[jax.ShapeDtypeStruct((G, H), jnp.float32),
               jax.ShapeDtypeStruct((G, C), jnp.float32)],
    scratch_shapes=[pltpu.VMEM((G, H), jnp.float32),
                    pltpu.VMEM((G, 1), jnp.float32)],
)


def kernel(x, edge_index, batch, W1_rel, W1_root, b1,
           W2_rel, W2_root, b2, W3_rel, W3_root, b3, Wl, bl):
    src = edge_index[0].reshape(NS, CHT, K)
    dst = edge_index[1].reshape(NS, CHT, K)
    zeros = jnp.zeros((ZR, HH), jnp.float32)
    batch3 = batch.reshape(RB, 1, BN)

    sc_scatter = _make_sc_scatter()
    m1, r1 = _lin2(x, W1_rel, W1_root, b1.reshape(1, H))
    p = sc_scatter(m1, src, dst, zeros)
    m2, r2 = _comb_lin2(p, r1, W2_rel, W2_root, b2.reshape(1, H))
    p = sc_scatter(m2, src, dst, zeros)
    m3, r3 = _comb_lin2(p, r2, W3_rel, W3_root, b3.reshape(1, H))
    p = sc_scatter(m3, src, dst, zeros)
    pooled, out = _pool(p, r3, batch3, Wl, bl.reshape(1, C))
    return (pooled, out)


# speedup vs baseline: 10.2639x; 1.0839x over previous
import functools

import jax
import jax.numpy as jnp
from jax import lax
from jax.experimental import pallas as pl
from jax.experimental.pallas import tpu as pltpu
from jax.experimental.pallas import tpu_sc as plsc

N = 10000
E = 320000
H = 128
G = 64
C = 10

NC = 2
NS = 16
HH = H // NC
EPT = E // NS
K = 125
CHT = EPT // K
RB = 10
BN = N // RB
NP = N
RPT = NP // NS
ZR = 125


NB = 6
LA = NB // 2
MAIN = (CHT - 2 * LA) // NB * NB


def _scatter_body(m_hbm, src_hbm, dst_hbm, zeros_hbm, out_hbm, *scr):
    src_v, dst_v = scr[0], scr[1]
    rows = scr[2:2 + NB]
    agg = scr[2 + NB]
    gsem = scr[3 + NB:3 + 2 * NB]
    ssem = scr[3 + 2 * NB:3 + 3 * NB]
    c = lax.axis_index("c")
    s = lax.axis_index("s")

    isrc = pltpu.make_async_copy(src_hbm.at[s], src_v, gsem[0])
    idst = pltpu.make_async_copy(dst_hbm.at[s], dst_v, gsem[1])
    isrc.start()
    idst.start()

    pltpu.sync_copy(zeros_hbm, rows[0])
    for j in range(RPT // ZR):
        pltpu.sync_copy(rows[0], agg.at[pl.ds(s * RPT + j * ZR, ZR)])
    isrc.wait()
    idst.wait()
    plsc.subcore_barrier()

    plane = m_hbm.at[c]

    def gather(i, b):
        return pltpu.make_async_copy(plane.at[src_v.at[i]], rows[b], gsem[b])

    def scatter(i, b):
        return pltpu.make_async_copy(rows[b], agg.at[dst_v.at[i]], ssem[b])

    for i in range(LA):
        gather(i, i % NB).start()
    for i in range(LA):
        gather(i, i % NB).wait()
        scatter(i, i % NB).start(add=True)
        gather(i + LA, (i + LA) % NB).start()

    def body(it, _):
        for b in range(NB):
            i = LA + NB * it + b
            bb = (LA + b) % NB
            gather(i, bb).wait()
            scatter(i, bb).start(add=True)
            scatter(i - LA, (bb + LA) % NB).wait()
            gather(i + LA, (bb + LA) % NB).start()
        return _

    lax.fori_loop(0, MAIN // NB, body, 0)
    for i in range(LA + MAIN, CHT):
        gather(i, i % NB).wait()
        scatter(i, i % NB).start(add=True)
        scatter(i - LA, (i - LA) % NB).wait()
        if i + LA < CHT:
            gather(i + LA, (i + LA) % NB).start()
    for i in range(CHT - LA, CHT):
        scatter(i, i % NB).wait()
    plsc.subcore_barrier()

    for j in range(RPT // ZR):
        row0 = s * RPT + j * ZR
        pltpu.sync_copy(agg.at[pl.ds(row0, ZR)], rows[j])
        pltpu.make_async_copy(rows[j], out_hbm.at[c, pl.ds(row0, ZR)],
                              ssem[j]).start()
    for j in range(RPT // ZR):
        pltpu.make_async_copy(rows[j], out_hbm.at[c, pl.ds(s * RPT + j * ZR, ZR)],
                              ssem[j]).wait()


@functools.lru_cache(maxsize=None)
def _make_sc_scatter():
    return pl.kernel(
        _scatter_body,
        out_type=jax.ShapeDtypeStruct((NC, NP, HH), jnp.float32),
        mesh=plsc.VectorSubcoreMesh(core_axis_name="c", subcore_axis_name="s",
                                    num_cores=NC, num_subcores=NS),
        scratch_types=(
            [pltpu.VMEM((CHT, K), jnp.int32)] * 2
            + [pltpu.VMEM((K, HH), jnp.float32)] * NB
            + [pltpu.VMEM_SHARED((NP, HH), jnp.float32)]
            + [pltpu.SemaphoreType.DMA] * (2 * NB)
        ),
        compiler_params=pltpu.CompilerParams(use_tc_tiling_on_sc=False),
    )


def _lin2_body(h_ref, wr_ref, wo_ref, b_ref, m_ref, r_ref):
    h = h_ref[...]
    m = jnp.dot(h, wr_ref[...], preferred_element_type=jnp.float32)
    m_ref[0] = m[:, :HH]
    m_ref[1] = m[:, HH:]
    r_ref[...] = (jnp.dot(h, wo_ref[...], preferred_element_type=jnp.float32)
                  + b_ref[...])


_lin2 = pl.pallas_call(
    _lin2_body,
    grid=(RB,),
    in_specs=[
        pl.BlockSpec((BN, H), lambda i: (i, 0)),
        pl.BlockSpec((H, H), lambda i: (0, 0)),
        pl.BlockSpec((H, H), lambda i: (0, 0)),
        pl.BlockSpec((1, H), lambda i: (0, 0)),
    ],
    out_specs=[pl.BlockSpec((NC, BN, HH), lambda i: (0, i, 0)),
               pl.BlockSpec((BN, H), lambda i: (i, 0))],
    out_shape=[jax.ShapeDtypeStruct((NC, N, HH), jnp.float32),
               jax.ShapeDtypeStruct((N, H), jnp.float32)],
)


def _comb_lin2_body(p_ref, rp_ref, wr_ref, wo_ref, b_ref, m_ref, r_ref):
    agg = jnp.concatenate([p_ref[0], p_ref[1]], axis=1)
    h = jnp.maximum(agg + rp_ref[...], 0.0)
    m = jnp.dot(h, wr_ref[...], preferred_element_type=jnp.float32)
    m_ref[0] = m[:, :HH]
    m_ref[1] = m[:, HH:]
    r_ref[...] = (jnp.dot(h, wo_ref[...], preferred_element_type=jnp.float32)
                  + b_ref[...])


_comb_lin2 = pl.pallas_call(
    _comb_lin2_body,
    grid=(RB,),
    in_specs=[
        pl.BlockSpec((NC, BN, HH), lambda i: (0, i, 0)),
        pl.BlockSpec((BN, H), lambda i: (i, 0)),
        pl.BlockSpec((H, H), lambda i: (0, 0)),
        pl.BlockSpec((H, H), lambda i: (0, 0)),
        pl.BlockSpec((1, H), lambda i: (0, 0)),
    ],
    out_specs=[pl.BlockSpec((NC, BN, HH), lambda i: (0, i, 0)),
               pl.BlockSpec((BN, H), lambda i: (i, 0))],
    out_shape=[jax.ShapeDtypeStruct((NC, N, HH), jnp.float32),
               jax.ShapeDtypeStruct((N, H), jnp.float32)],
)


def _pool_body(p_ref, rp_ref, batch_ref, wl_ref, bl_ref,
               pooled_ref, out_ref, sum_acc, cnt_acc):
    i = pl.program_id(0)
    agg = jnp.concatenate([p_ref[0], p_ref[1]], axis=1)
    h = agg + rp_ref[...]
    b_row = batch_ref[0]
    oh_t = (lax.broadcasted_iota(jnp.int32, (G, BN), 0) == b_row
            ).astype(jnp.float32)

    @pl.when(i == 0)
    def _():
        sum_acc[...] = jnp.zeros_like(sum_acc)
        cnt_acc[...] = jnp.zeros_like(cnt_acc)

    sum_acc[...] += lax.dot_general(oh_t, h, (((1,), (0,)), ((), ())),
                                    preferred_element_type=jnp.float32)
    cnt_acc[...] += jnp.sum(oh_t, axis=1)[:, None]

    @pl.when(i == RB - 1)
    def _():
        pooled = sum_acc[...] / jnp.maximum(cnt_acc[...], 1.0)
        pooled_ref[...] = pooled
        out_ref[...] = (jnp.dot(pooled, wl_ref[...],
                                preferred_element_type=jnp.float32)
                        + bl_ref[...])


_pool = pl.pallas_call(
    _pool_body,
    grid=(RB,),
    in_specs=[
        pl.BlockSpec((NC, BN, HH), lambda i: (0, i, 0)),
        pl.BlockSpec((BN, H), lambda i: (i, 0)),
        pl.BlockSpec((1, 1, BN), lambda i: (i, 0, 0)),
        pl.BlockSpec((H, C), lambda i: (0, 0)),
        pl.BlockSpec((1, C), lambda i: (0, 0)),
    ],
    out_specs=[pl.BlockSpec((G, H), lambda i: (0, 0)),
               pl.BlockSpec((G, C), lambda i: (0, 0))],
    out_shape=[jax.ShapeDtypeStruct((G, H), jnp.float32),
               jax.ShapeDtypeStruct((G, C), jnp.float32)],
    scratch_shapes=[pltpu.VMEM((G, H), jnp.float32),
                    pltpu.VMEM((G, 1), jnp.float32)],
)


def kernel(x, edge_index, batch, W1_rel, W1_root, b1,
           W2_rel, W2_root, b2, W3_rel, W3_root, b3, Wl, bl):
    src = edge_index[0].reshape(NS, CHT, K)
    dst = edge_index[1].reshape(NS, CHT, K)
    zeros = jnp.zeros((ZR, HH), jnp.float32)
    batch3 = batch.reshape(RB, 1, BN)

    sc_scatter = _make_sc_scatter()
    m1, r1 = _lin2(x, W1_rel, W1_root, b1.reshape(1, H))
    p = sc_scatter(m1, src, dst, zeros)
    m2, r2 = _comb_lin2(p, r1, W2_rel, W2_root, b2.reshape(1, H))
    p = sc_scatter(m2, src, dst, zeros)
    m3, r3 = _comb_lin2(p, r2, W3_rel, W3_root, b3.reshape(1, H))
    p = sc_scatter(m3, src, dst, zeros)
    pooled, out = _pool(p, r3, batch3, Wl, bl.reshape(1, C))
    return (pooled, out)


# speedup vs baseline: 10.5261x; 1.0255x over previous
import functools

import jax
import jax.numpy as jnp
from jax import lax
from jax.experimental import pallas as pl
from jax.experimental.pallas import tpu as pltpu
from jax.experimental.pallas import tpu_sc as plsc

N = 10000
E = 320000
H = 128
G = 64
C = 10

NC = 2
NS = 16
HH = H // NC
EPT = E // NS
K = 125
CHT = EPT // K
RB = 10
BN = N // RB
NP = N
RPT = NP // NS
ZR = 125


NB = 6
LA = NB // 2
MAIN = (CHT - 2 * LA) // NB * NB


def _scatter_body(m_hbm, ei_hbm, zeros_hbm, out_hbm, *scr):
    src_v, dst_v = scr[0], scr[1]
    rows = scr[2:2 + NB]
    agg = scr[2 + NB]
    gsem = scr[3 + NB:3 + 2 * NB]
    ssem = scr[3 + 2 * NB:3 + 3 * NB]
    c = lax.axis_index("c")
    s = lax.axis_index("s")

    isrc = pltpu.make_async_copy(ei_hbm.at[0].at[s], src_v, gsem[0])
    idst = pltpu.make_async_copy(ei_hbm.at[1].at[s], dst_v, gsem[1])
    isrc.start()
    idst.start()

    pltpu.sync_copy(zeros_hbm, rows[0])
    for j in range(RPT // ZR):
        pltpu.sync_copy(rows[0], agg.at[pl.ds(s * RPT + j * ZR, ZR)])
    isrc.wait()
    idst.wait()
    plsc.subcore_barrier()

    plane = m_hbm.at[c]

    def gather(i, b):
        return pltpu.make_async_copy(plane.at[src_v.at[i]], rows[b], gsem[b])

    def scatter(i, b):
        return pltpu.make_async_copy(rows[b], agg.at[dst_v.at[i]], ssem[b])

    for i in range(LA):
        gather(i, i % NB).start()
    for i in range(LA):
        gather(i, i % NB).wait()
        scatter(i, i % NB).start(add=True)
        gather(i + LA, (i + LA) % NB).start()

    def body(it, _):
        for b in range(NB):
            i = LA + NB * it + b
            bb = (LA + b) % NB
            gather(i, bb).wait()
            scatter(i, bb).start(add=True)
            scatter(i - LA, (bb + LA) % NB).wait()
            gather(i + LA, (bb + LA) % NB).start()
        return _

    lax.fori_loop(0, MAIN // NB, body, 0)
    for i in range(LA + MAIN, CHT):
        gather(i, i % NB).wait()
        scatter(i, i % NB).start(add=True)
        scatter(i - LA, (i - LA) % NB).wait()
        if i + LA < CHT:
            gather(i + LA, (i + LA) % NB).start()
    for i in range(CHT - LA, CHT):
        scatter(i, i % NB).wait()
    plsc.subcore_barrier()

    for j in range(RPT // ZR):
        row0 = s * RPT + j * ZR
        pltpu.sync_copy(agg.at[pl.ds(row0, ZR)], rows[j])
        pltpu.make_async_copy(rows[j], out_hbm.at[c, pl.ds(row0, ZR)],
                              ssem[j]).start()
    for j in range(RPT // ZR):
        pltpu.make_async_copy(rows[j], out_hbm.at[c, pl.ds(s * RPT + j * ZR, ZR)],
                              ssem[j]).wait()


@functools.lru_cache(maxsize=None)
def _make_sc_scatter():
    return pl.kernel(
        _scatter_body,
        out_type=jax.ShapeDtypeStruct((NC, NP, HH), jnp.float32),
        mesh=plsc.VectorSubcoreMesh(core_axis_name="c", subcore_axis_name="s",
                                    num_cores=NC, num_subcores=NS),
        scratch_types=(
            [pltpu.VMEM((CHT, K), jnp.int32)] * 2
            + [pltpu.VMEM((K, HH), jnp.float32)] * NB
            + [pltpu.VMEM_SHARED((NP, HH), jnp.float32)]
            + [pltpu.SemaphoreType.DMA] * (2 * NB)
        ),
        compiler_params=pltpu.CompilerParams(use_tc_tiling_on_sc=False),
    )


def _lin2_body(h_ref, wr_ref, wo_ref, b_ref, m_ref, r_ref):
    h = h_ref[...]
    m = jnp.dot(h, wr_ref[...], preferred_element_type=jnp.float32)
    m_ref[0] = m[:, :HH]
    m_ref[1] = m[:, HH:]
    r_ref[...] = (jnp.dot(h, wo_ref[...], preferred_element_type=jnp.float32)
                  + b_ref[...])


_lin2 = pl.pallas_call(
    _lin2_body,
    grid=(RB,),
    in_specs=[
        pl.BlockSpec((BN, H), lambda i: (i, 0)),
        pl.BlockSpec((H, H), lambda i: (0, 0)),
        pl.BlockSpec((H, H), lambda i: (0, 0)),
        pl.BlockSpec((1, H), lambda i: (0, 0)),
    ],
    out_specs=[pl.BlockSpec((NC, BN, HH), lambda i: (0, i, 0)),
               pl.BlockSpec((BN, H), lambda i: (i, 0))],
    out_shape=[jax.ShapeDtypeStruct((NC, N, HH), jnp.float32),
               jax.ShapeDtypeStruct((N, H), jnp.float32)],
)


def _comb_lin2_body(p_ref, rp_ref, wr_ref, wo_ref, b_ref, m_ref, r_ref):
    agg = jnp.concatenate([p_ref[0], p_ref[1]], axis=1)
    h = jnp.maximum(agg + rp_ref[...], 0.0)
    m = jnp.dot(h, wr_ref[...], preferred_element_type=jnp.float32)
    m_ref[0] = m[:, :HH]
    m_ref[1] = m[:, HH:]
    r_ref[...] = (jnp.dot(h, wo_ref[...], preferred_element_type=jnp.float32)
                  + b_ref[...])


_comb_lin2 = pl.pallas_call(
    _comb_lin2_body,
    grid=(RB,),
    in_specs=[
        pl.BlockSpec((NC, BN, HH), lambda i: (0, i, 0)),
        pl.BlockSpec((BN, H), lambda i: (i, 0)),
        pl.BlockSpec((H, H), lambda i: (0, 0)),
        pl.BlockSpec((H, H), lambda i: (0, 0)),
        pl.BlockSpec((1, H), lambda i: (0, 0)),
    ],
    out_specs=[pl.BlockSpec((NC, BN, HH), lambda i: (0, i, 0)),
               pl.BlockSpec((BN, H), lambda i: (i, 0))],
    out_shape=[jax.ShapeDtypeStruct((NC, N, HH), jnp.float32),
               jax.ShapeDtypeStruct((N, H), jnp.float32)],
)


def _pool_body(p_ref, rp_ref, batch_ref, wl_ref, bl_ref,
               pooled_ref, out_ref, sum_acc, cnt_acc):
    i = pl.program_id(0)
    agg = jnp.concatenate([p_ref[0], p_ref[1]], axis=1)
    h = agg + rp_ref[...]
    b_row = batch_ref[0]
    oh_t = (lax.broadcasted_iota(jnp.int32, (G, BN), 0) == b_row
            ).astype(jnp.float32)

    @pl.when(i == 0)
    def _():
        sum_acc[...] = jnp.zeros_like(sum_acc)
        cnt_acc[...] = jnp.zeros_like(cnt_acc)

    sum_acc[...] += lax.dot_general(oh_t, h, (((1,), (0,)), ((), ())),
                                    preferred_element_type=jnp.float32)
    cnt_acc[...] += jnp.sum(oh_t, axis=1)[:, None]

    @pl.when(i == RB - 1)
    def _():
        pooled = sum_acc[...] / jnp.maximum(cnt_acc[...], 1.0)
        pooled_ref[...] = pooled
        out_ref[...] = (jnp.dot(pooled, wl_ref[...],
                                preferred_element_type=jnp.float32)
                        + bl_ref[...])


_pool = pl.pallas_call(
    _pool_body,
    grid=(RB,),
    in_specs=[
        pl.BlockSpec((NC, BN, HH), lambda i: (0, i, 0)),
        pl.BlockSpec((BN, H), lambda i: (i, 0)),
        pl.BlockSpec((1, 1, BN), lambda i: (i, 0, 0)),
        pl.BlockSpec((H, C), lambda i: (0, 0)),
        pl.BlockSpec((1, C), lambda i: (0, 0)),
    ],
    out_specs=[pl.BlockSpec((G, H), lambda i: (0, 0)),
               pl.BlockSpec((G, C), lambda i: (0, 0))],
    out_shape=[jax.ShapeDtypeStruct((G, H), jnp.float32),
               jax.ShapeDtypeStruct((G, C), jnp.float32)],
    scratch_shapes=[pltpu.VMEM((G, H), jnp.float32),
                    pltpu.VMEM((G, 1), jnp.float32)],
)


def kernel(x, edge_index, batch, W1_rel, W1_root, b1,
           W2_rel, W2_root, b2, W3_rel, W3_root, b3, Wl, bl):
    ei4 = edge_index.reshape(2, NS, CHT, K)
    zeros = jnp.zeros((ZR, HH), jnp.float32)
    batch3 = batch.reshape(RB, 1, BN)

    sc_scatter = _make_sc_scatter()
    m1, r1 = _lin2(x, W1_rel, W1_root, b1.reshape(1, H))
    p = sc_scatter(m1, ei4, zeros)
    m2, r2 = _comb_lin2(p, r1, W2_rel, W2_root, b2.reshape(1, H))
    p = sc_scatter(m2, ei4, zeros)
    m3, r3 = _comb_lin2(p, r2, W3_rel, W3_root, b3.reshape(1, H))
    p = sc_scatter(m3, ei4, zeros)
    pooled, out = _pool(p, r3, batch3, Wl, bl.reshape(1, C))
    return (pooled, out)


# speedup vs baseline: 10.7893x; 1.0250x over previous
import functools

import jax
import jax.numpy as jnp
from jax import lax
from jax.experimental import pallas as pl
from jax.experimental.pallas import tpu as pltpu
from jax.experimental.pallas import tpu_sc as plsc

N = 10000
E = 320000
H = 128
G = 64
C = 10

NC = 2
NS = 16
HH = H // NC
EPT = E // NS
K = 125
CHT = EPT // K
RB = 5
BN = N // RB
NP = N
RPT = NP // NS
ZR = 125


NB = 6
LA = NB // 2
MAIN = (CHT - 2 * LA) // NB * NB


def _scatter_body(m_hbm, ei_hbm, zeros_hbm, out_hbm, *scr):
    src_v, dst_v = scr[0], scr[1]
    rows = scr[2:2 + NB]
    agg = scr[2 + NB]
    gsem = scr[3 + NB:3 + 2 * NB]
    ssem = scr[3 + 2 * NB:3 + 3 * NB]
    c = lax.axis_index("c")
    s = lax.axis_index("s")

    isrc = pltpu.make_async_copy(ei_hbm.at[0].at[s], src_v, gsem[0])
    idst = pltpu.make_async_copy(ei_hbm.at[1].at[s], dst_v, gsem[1])
    isrc.start()
    idst.start()

    pltpu.sync_copy(zeros_hbm, rows[0])
    for j in range(RPT // ZR):
        pltpu.sync_copy(rows[0], agg.at[pl.ds(s * RPT + j * ZR, ZR)])
    isrc.wait()
    idst.wait()
    plsc.subcore_barrier()

    plane = m_hbm.at[c]

    def gather(i, b):
        return pltpu.make_async_copy(plane.at[src_v.at[i]], rows[b], gsem[b])

    def scatter(i, b):
        return pltpu.make_async_copy(rows[b], agg.at[dst_v.at[i]], ssem[b])

    for i in range(LA):
        gather(i, i % NB).start()
    for i in range(LA):
        gather(i, i % NB).wait()
        scatter(i, i % NB).start(add=True)
        gather(i + LA, (i + LA) % NB).start()

    def body(it, _):
        for b in range(NB):
            i = LA + NB * it + b
            bb = (LA + b) % NB
            gather(i, bb).wait()
            scatter(i, bb).start(add=True)
            scatter(i - LA, (bb + LA) % NB).wait()
            gather(i + LA, (bb + LA) % NB).start()
        return _

    lax.fori_loop(0, MAIN // NB, body, 0)
    for i in range(LA + MAIN, CHT):
        gather(i, i % NB).wait()
        scatter(i, i % NB).start(add=True)
        scatter(i - LA, (i - LA) % NB).wait()
        if i + LA < CHT:
            gather(i + LA, (i + LA) % NB).start()
    for i in range(CHT - LA, CHT):
        scatter(i, i % NB).wait()
    plsc.subcore_barrier()

    for j in range(RPT // ZR):
        row0 = s * RPT + j * ZR
        pltpu.sync_copy(agg.at[pl.ds(row0, ZR)], rows[j])
        pltpu.make_async_copy(rows[j], out_hbm.at[c, pl.ds(row0, ZR)],
                              ssem[j]).start()
    for j in range(RPT // ZR):
        pltpu.make_async_copy(rows[j], out_hbm.at[c, pl.ds(s * RPT + j * ZR, ZR)],
                              ssem[j]).wait()


@functools.lru_cache(maxsize=None)
def _make_sc_scatter():
    return pl.kernel(
        _scatter_body,
        out_type=jax.ShapeDtypeStruct((NC, NP, HH), jnp.float32),
        mesh=plsc.VectorSubcoreMesh(core_axis_name="c", subcore_axis_name="s",
                                    num_cores=NC, num_subcores=NS),
        scratch_types=(
            [pltpu.VMEM((CHT, K), jnp.int32)] * 2
            + [pltpu.VMEM((K, HH), jnp.float32)] * NB
            + [pltpu.VMEM_SHARED((NP, HH), jnp.float32)]
            + [pltpu.SemaphoreType.DMA] * (2 * NB)
        ),
        compiler_params=pltpu.CompilerParams(use_tc_tiling_on_sc=False),
    )


def _lin2_body(h_ref, wr_ref, wo_ref, b_ref, m_ref, r_ref):
    h = h_ref[...]
    m = jnp.dot(h, wr_ref[...], preferred_element_type=jnp.float32)
    m_ref[0] = m[:, :HH]
    m_ref[1] = m[:, HH:]
    r_ref[...] = (jnp.dot(h, wo_ref[...], preferred_element_type=jnp.float32)
                  + b_ref[...])


_lin2 = pl.pallas_call(
    _lin2_body,
    grid=(RB,),
    in_specs=[
        pl.BlockSpec((BN, H), lambda i: (i, 0)),
        pl.BlockSpec((H, H), lambda i: (0, 0)),
        pl.BlockSpec((H, H), lambda i: (0, 0)),
        pl.BlockSpec((1, H), lambda i: (0, 0)),
    ],
    out_specs=[pl.BlockSpec((NC, BN, HH), lambda i: (0, i, 0)),
               pl.BlockSpec((BN, H), lambda i: (i, 0))],
    out_shape=[jax.ShapeDtypeStruct((NC, N, HH), jnp.float32),
               jax.ShapeDtypeStruct((N, H), jnp.float32)],
)


def _comb_lin2_body(p_ref, rp_ref, wr_ref, wo_ref, b_ref, m_ref, r_ref):
    agg = jnp.concatenate([p_ref[0], p_ref[1]], axis=1)
    h = jnp.maximum(agg + rp_ref[...], 0.0)
    m = jnp.dot(h, wr_ref[...], preferred_element_type=jnp.float32)
    m_ref[0] = m[:, :HH]
    m_ref[1] = m[:, HH:]
    r_ref[...] = (jnp.dot(h, wo_ref[...], preferred_element_type=jnp.float32)
                  + b_ref[...])


_comb_lin2 = pl.pallas_call(
    _comb_lin2_body,
    grid=(RB,),
    in_specs=[
        pl.BlockSpec((NC, BN, HH), lambda i: (0, i, 0)),
        pl.BlockSpec((BN, H), lambda i: (i, 0)),
        pl.BlockSpec((H, H), lambda i: (0, 0)),
        pl.BlockSpec((H, H), lambda i: (0, 0)),
        pl.BlockSpec((1, H), lambda i: (0, 0)),
    ],
    out_specs=[pl.BlockSpec((NC, BN, HH), lambda i: (0, i, 0)),
               pl.BlockSpec((BN, H), lambda i: (i, 0))],
    out_shape=[jax.ShapeDtypeStruct((NC, N, HH), jnp.float32),
               jax.ShapeDtypeStruct((N, H), jnp.float32)],
)


def _pool_body(p_ref, rp_ref, batch_ref, wl_ref, bl_ref,
               pooled_ref, out_ref, sum_acc, cnt_acc):
    i = pl.program_id(0)
    agg = jnp.concatenate([p_ref[0], p_ref[1]], axis=1)
    h = agg + rp_ref[...]
    b_row = batch_ref[0]
    oh_t = (lax.broadcasted_iota(jnp.int32, (G, BN), 0) == b_row
            ).astype(jnp.float32)

    @pl.when(i == 0)
    def _():
        sum_acc[...] = jnp.zeros_like(sum_acc)
        cnt_acc[...] = jnp.zeros_like(cnt_acc)

    sum_acc[...] += lax.dot_general(oh_t, h, (((1,), (0,)), ((), ())),
                                    preferred_element_type=jnp.float32)
    cnt_acc[...] += jnp.sum(oh_t, axis=1)[:, None]

    @pl.when(i == RB - 1)
    def _():
        pooled = sum_acc[...] / jnp.maximum(cnt_acc[...], 1.0)
        pooled_ref[...] = pooled
        out_ref[...] = (jnp.dot(pooled, wl_ref[...],
                                preferred_element_type=jnp.float32)
                        + bl_ref[...])


_pool = pl.pallas_call(
    _pool_body,
    grid=(RB,),
    in_specs=[
        pl.BlockSpec((NC, BN, HH), lambda i: (0, i, 0)),
        pl.BlockSpec((BN, H), lambda i: (i, 0)),
        pl.BlockSpec((1, 1, BN), lambda i: (i, 0, 0)),
        pl.BlockSpec((H, C), lambda i: (0, 0)),
        pl.BlockSpec((1, C), lambda i: (0, 0)),
    ],
    out_specs=[pl.BlockSpec((G, H), lambda i: (0, 0)),
               pl.BlockSpec((G, C), lambda i: (0, 0))],
    out_shape=[jax.ShapeDtypeStruct((G, H), jnp.float32),
               jax.ShapeDtypeStruct((G, C), jnp.float32)],
    scratch_shapes=[pltpu.VMEM((G, H), jnp.float32),
                    pltpu.VMEM((G, 1), jnp.float32)],
)


def kernel(x, edge_index, batch, W1_rel, W1_root, b1,
           W2_rel, W2_root, b2, W3_rel, W3_root, b3, Wl, bl):
    ei4 = edge_index.reshape(2, NS, CHT, K)
    zeros = jnp.zeros((ZR, HH), jnp.float32)
    batch3 = batch.reshape(RB, 1, BN)

    sc_scatter = _make_sc_scatter()
    m1, r1 = _lin2(x, W1_rel, W1_root, b1.reshape(1, H))
    p = sc_scatter(m1, ei4, zeros)
    m2, r2 = _comb_lin2(p, r1, W2_rel, W2_root, b2.reshape(1, H))
    p = sc_scatter(m2, ei4, zeros)
    m3, r3 = _comb_lin2(p, r2, W3_rel, W3_root, b3.reshape(1, H))
    p = sc_scatter(m3, ei4, zeros)
    pooled, out = _pool(p, r3, batch3, Wl, bl.reshape(1, C))
    return (pooled, out)


# speedup vs baseline: 11.0320x; 1.0225x over previous
import functools

import jax
import jax.numpy as jnp
from jax import lax
from jax.experimental import pallas as pl
from jax.experimental.pallas import tpu as pltpu
from jax.experimental.pallas import tpu_sc as plsc

N = 10000
E = 320000
H = 128
G = 64
C = 10

NC = 2
NS = 16
HH = H // NC
EPT = E // NS
K = 80
CHT = EPT // K
RB = 5
BN = N // RB
NP = N
RPT = NP // NS
ZR = 80
DR_CHUNKS = [(j * ZR, ZR) for j in range(7)] + [(7 * ZR, RPT - 7 * ZR)]

NB = 8
LA = NB // 2
MAIN = (CHT - 2 * LA) // NB * NB


def _scatter_body(m_hbm, ei_hbm, zeros_hbm, out_hbm, *scr):
    src_v, dst_v = scr[0], scr[1]
    rows = scr[2:2 + NB]
    agg = scr[2 + NB]
    gsem = scr[3 + NB:3 + 2 * NB]
    ssem = scr[3 + 2 * NB:3 + 3 * NB]
    c = lax.axis_index("c")
    s = lax.axis_index("s")

    isrc = pltpu.make_async_copy(ei_hbm.at[0].at[s], src_v, gsem[0])
    idst = pltpu.make_async_copy(ei_hbm.at[1].at[s], dst_v, gsem[1])
    isrc.start()
    idst.start()

    pltpu.sync_copy(zeros_hbm, rows[0])
    for off, sz in DR_CHUNKS:
        pltpu.sync_copy(rows[0].at[pl.ds(0, sz)],
                        agg.at[pl.ds(s * RPT + off, sz)])
    isrc.wait()
    idst.wait()
    plsc.subcore_barrier()

    plane = m_hbm.at[c]

    def gather(i, b):
        return pltpu.make_async_copy(plane.at[src_v.at[i]], rows[b], gsem[b])

    def scatter(i, b):
        return pltpu.make_async_copy(rows[b], agg.at[dst_v.at[i]], ssem[b])

    for i in range(LA):
        gather(i, i % NB).start()
    for i in range(LA):
        gather(i, i % NB).wait()
        scatter(i, i % NB).start(add=True)
        gather(i + LA, (i + LA) % NB).start()

    def body(it, _):
        for b in range(NB):
            i = LA + NB * it + b
            bb = (LA + b) % NB
            gather(i, bb).wait()
            scatter(i, bb).start(add=True)
            scatter(i - LA, (bb + LA) % NB).wait()
            gather(i + LA, (bb + LA) % NB).start()
        return _

    lax.fori_loop(0, MAIN // NB, body, 0)
    for i in range(LA + MAIN, CHT):
        gather(i, i % NB).wait()
        scatter(i, i % NB).start(add=True)
        scatter(i - LA, (i - LA) % NB).wait()
        if i + LA < CHT:
            gather(i + LA, (i + LA) % NB).start()
    for i in range(CHT - LA, CHT):
        scatter(i, i % NB).wait()
    plsc.subcore_barrier()

    for j, (off, sz) in enumerate(DR_CHUNKS):
        row0 = s * RPT + off
        pltpu.sync_copy(agg.at[pl.ds(row0, sz)], rows[j].at[pl.ds(0, sz)])
        pltpu.make_async_copy(rows[j].at[pl.ds(0, sz)],
                              out_hbm.at[c, pl.ds(row0, sz)],
                              ssem[j]).start()
    for j, (off, sz) in enumerate(DR_CHUNKS):
        pltpu.make_async_copy(rows[j].at[pl.ds(0, sz)],
                              out_hbm.at[c, pl.ds(s * RPT + off, sz)],
                              ssem[j]).wait()


@functools.lru_cache(maxsize=None)
def _make_sc_scatter():
    return pl.kernel(
        _scatter_body,
        out_type=jax.ShapeDtypeStruct((NC, NP, HH), jnp.float32),
        mesh=plsc.VectorSubcoreMesh(core_axis_name="c", subcore_axis_name="s",
                                    num_cores=NC, num_subcores=NS),
        scratch_types=(
            [pltpu.VMEM((CHT, K), jnp.int32)] * 2
            + [pltpu.VMEM((K, HH), jnp.float32)] * NB
            + [pltpu.VMEM_SHARED((NP, HH), jnp.float32)]
            + [pltpu.SemaphoreType.DMA] * (2 * NB)
        ),
        compiler_params=pltpu.CompilerParams(use_tc_tiling_on_sc=False),
    )


def _lin2_body(h_ref, wr_ref, wo_ref, b_ref, m_ref, r_ref):
    h = h_ref[...]
    m = jnp.dot(h, wr_ref[...], preferred_element_type=jnp.float32)
    m_ref[0] = m[:, :HH]
    m_ref[1] = m[:, HH:]
    r_ref[...] = (jnp.dot(h, wo_ref[...], preferred_element_type=jnp.float32)
                  + b_ref[...])


_lin2 = pl.pallas_call(
    _lin2_body,
    grid=(RB,),
    in_specs=[
        pl.BlockSpec((BN, H), lambda i: (i, 0)),
        pl.BlockSpec((H, H), lambda i: (0, 0)),
        pl.BlockSpec((H, H), lambda i: (0, 0)),
        pl.BlockSpec((1, H), lambda i: (0, 0)),
    ],
    out_specs=[pl.BlockSpec((NC, BN, HH), lambda i: (0, i, 0)),
               pl.BlockSpec((BN, H), lambda i: (i, 0))],
    out_shape=[jax.ShapeDtypeStruct((NC, N, HH), jnp.float32),
               jax.ShapeDtypeStruct((N, H), jnp.float32)],
)


def _comb_lin2_body(p_ref, rp_ref, wr_ref, wo_ref, b_ref, m_ref, r_ref):
    agg = jnp.concatenate([p_ref[0], p_ref[1]], axis=1)
    h = jnp.maximum(agg + rp_ref[...], 0.0)
    m = jnp.dot(h, wr_ref[...], preferred_element_type=jnp.float32)
    m_ref[0] = m[:, :HH]
    m_ref[1] = m[:, HH:]
    r_ref[...] = (jnp.dot(h, wo_ref[...], preferred_element_type=jnp.float32)
                  + b_ref[...])


_comb_lin2 = pl.pallas_call(
    _comb_lin2_body,
    grid=(RB,),
    in_specs=[
        pl.BlockSpec((NC, BN, HH), lambda i: (0, i, 0)),
        pl.BlockSpec((BN, H), lambda i: (i, 0)),
        pl.BlockSpec((H, H), lambda i: (0, 0)),
        pl.BlockSpec((H, H), lambda i: (0, 0)),
        pl.BlockSpec((1, H), lambda i: (0, 0)),
    ],
    out_specs=[pl.BlockSpec((NC, BN, HH), lambda i: (0, i, 0)),
               pl.BlockSpec((BN, H), lambda i: (i, 0))],
    out_shape=[jax.ShapeDtypeStruct((NC, N, HH), jnp.float32),
               jax.ShapeDtypeStruct((N, H), jnp.float32)],
)


def _pool_body(p_ref, rp_ref, batch_ref, wl_ref, bl_ref,
               pooled_ref, out_ref, sum_acc, cnt_acc):
    i = pl.program_id(0)
    agg = jnp.concatenate([p_ref[0], p_ref[1]], axis=1)
    h = agg + rp_ref[...]
    b_row = batch_ref[0]
    oh_t = (lax.broadcasted_iota(jnp.int32, (G, BN), 0) == b_row
            ).astype(jnp.float32)

    @pl.when(i == 0)
    def _():
        sum_acc[...] = jnp.zeros_like(sum_acc)
        cnt_acc[...] = jnp.zeros_like(cnt_acc)

    sum_acc[...] += lax.dot_general(oh_t, h, (((1,), (0,)), ((), ())),
                                    preferred_element_type=jnp.float32)
    cnt_acc[...] += jnp.sum(oh_t, axis=1)[:, None]

    @pl.when(i == RB - 1)
    def _():
        pooled = sum_acc[...] / jnp.maximum(cnt_acc[...], 1.0)
        pooled_ref[...] = pooled
        out_ref[...] = (jnp.dot(pooled, wl_ref[...],
                                preferred_element_type=jnp.float32)
                        + bl_ref[...])


_pool = pl.pallas_call(
    _pool_body,
    grid=(RB,),
    in_specs=[
        pl.BlockSpec((NC, BN, HH), lambda i: (0, i, 0)),
        pl.BlockSpec((BN, H), lambda i: (i, 0)),
        pl.BlockSpec((1, 1, BN), lambda i: (i, 0, 0)),
        pl.BlockSpec((H, C), lambda i: (0, 0)),
        pl.BlockSpec((1, C), lambda i: (0, 0)),
    ],
    out_specs=[pl.BlockSpec((G, H), lambda i: (0, 0)),
               pl.BlockSpec((G, C), lambda i: (0, 0))],
    out_shape=[jax.ShapeDtypeStruct((G, H), jnp.float32),
               jax.ShapeDtypeStruct((G, C), jnp.float32)],
    scratch_shapes=[pltpu.VMEM((G, H), jnp.float32),
                    pltpu.VMEM((G, 1), jnp.float32)],
)


def kernel(x, edge_index, batch, W1_rel, W1_root, b1,
           W2_rel, W2_root, b2, W3_rel, W3_root, b3, Wl, bl):
    ei4 = edge_index.reshape(2, NS, CHT, K)
    zeros = jnp.zeros((ZR, HH), jnp.float32)
    batch3 = batch.reshape(RB, 1, BN)

    sc_scatter = _make_sc_scatter()
    m1, r1 = _lin2(x, W1_rel, W1_root, b1.reshape(1, H))
    p = sc_scatter(m1, ei4, zeros)
    m2, r2 = _comb_lin2(p, r1, W2_rel, W2_root, b2.reshape(1, H))
    p = sc_scatter(m2, ei4, zeros)
    m3, r3 = _comb_lin2(p, r2, W3_rel, W3_root, b3.reshape(1, H))
    p = sc_scatter(m3, ei4, zeros)
    pooled, out = _pool(p, r3, batch3, Wl, bl.reshape(1, C))
    return (pooled, out)
